# baseline (speedup 1.0000x reference)
import functools

import jax
import jax.numpy as jnp
import numpy as np
from jax import lax
from jax.experimental import pallas as pl
from jax.experimental.pallas import tpu as pltpu
from jax.experimental.pallas import tpu_sc as plsc

B, M, F, K, E = 8, 10000, 128, 3, 320000
FP = F // 2
NC, NS = 2, 16
EP = E // NS
G = 80
SG = 2000
CPS = SG // G
NSUP = EP // SG
DR = 624
TAIL = M - NS * DR
ZR = 78
BPC = B // NC

_PERM = np.empty(F, np.int32)
for _blk in range(F // 32):
    for _i in range(16):
        _PERM[32 * _blk + _i] = 32 * _blk + 2 * _i
        _PERM[32 * _blk + 16 + _i] = 32 * _blk + 2 * _i + 1
_PERM2 = _PERM[_PERM]


def _to_pairs(a):
    ab = a.astype(jnp.bfloat16).reshape(-1, FP, 2)
    return jax.lax.bitcast_convert_type(ab, jnp.int32)


def _spmm_sc(xpairs, src, dst, w):
    mesh = plsc.VectorSubcoreMesh(
        core_axis_name="c", subcore_axis_name="s", num_cores=NC, num_subcores=NS
    )

    @functools.partial(
        pl.kernel,
        out_type=jax.ShapeDtypeStruct((B * M, F), jnp.float32),
        mesh=mesh,
        compiler_params=pltpu.CompilerParams(use_tc_tiling_on_sc=False),
        scratch_types=[
            pltpu.VMEM_SHARED((M, F), jnp.float32),
            pltpu.VMEM((SG,), jnp.int32),
            pltpu.VMEM((SG,), jnp.int32),
            pltpu.VMEM((SG,), jnp.float32),
            pltpu.VMEM((SG,), jnp.int32),
            pltpu.VMEM((G, FP), jnp.int32),
            pltpu.VMEM((G, FP), jnp.int32),
            pltpu.VMEM((G, F), jnp.float32),
            pltpu.VMEM((G, F), jnp.float32),
            pltpu.VMEM((G, F), jnp.float32),
            pltpu.VMEM((G,), jnp.int32),
            pltpu.VMEM((G,), jnp.int32),
            pltpu.VMEM((G,), jnp.int32),
            pltpu.SemaphoreType.DMA,
            pltpu.SemaphoreType.DMA,
            pltpu.SemaphoreType.DMA,
            pltpu.SemaphoreType.DMA,
            pltpu.SemaphoreType.DMA,
        ],
    )
    def run(x_hbm, src_hbm, dst_hbm, w_hbm, y_hbm,
            acc, srcv, dstv, wv, gidx, rin0, rin1, rout0, rout1, rout2,
            d0, d1, d2, gsem0, gsem1, ssem0, ssem1, ssem2):
        cid = lax.axis_index("c")
        sid = lax.axis_index("s")
        ebase = sid * EP

        rin = (rin0, rin1)
        rout = (rout0, rout1, rout2)
        dbuf = (d0, d1, d2)
        gsems = (gsem0, gsem1)
        ssems = (ssem0, ssem1, ssem2)

        def gather(ci, p):
            pltpu.async_copy(x_hbm.at[gidx.at[pl.ds(ci * G, G)]], rin[p], gsems[p])

        def wait_gather(ci, p):
            pltpu.make_async_copy(
                x_hbm.at[gidx.at[pl.ds(ci * G, G)]], rin[p], gsems[p]
            ).wait()

        def wait_scatter(q):
            pltpu.make_async_copy(rout[q], acc.at[dbuf[q]], ssems[q]).wait()

        def process(ci, p, q):
            wait_gather(ci, p)
            for j in range(G // 16):
                dbuf[q][pl.ds(j * 16, 16)] = dstv[pl.ds(ci * G + j * 16, 16)]

            @pl.loop(0, G // 16)
            def _scale(g):
                wvec = wv[pl.ds(ci * G + g * 16, 16)]
                for l in range(16):
                    we = wvec[l]
                    e = g * 16 + l
                    for blk in range(F // 32):
                        v = rin[p][e, pl.ds(blk * 16, 16)]
                        va = jax.lax.bitcast_convert_type(v << 16, jnp.float32)
                        vb = jax.lax.bitcast_convert_type(
                            v & jnp.int32(-65536), jnp.float32
                        )
                        rout[q][e, pl.ds(blk * 32, 16)] = va * we
                        rout[q][e, pl.ds(blk * 32 + 16, 16)] = vb * we

            pltpu.async_copy(rout[q], acc.at[dbuf[q]], ssems[q], add=True)

        @pl.loop(0, BPC)
        def _batch(bi):
            b = cid * BPC + bi
            off = b * M

            @pl.loop(0, G)
            def _zfill(r):
                for j in range(F // 16):
                    rout0[r, pl.ds(j * 16, 16)] = jnp.zeros((16,), jnp.float32)

            for z in range(DR // ZR):
                pltpu.sync_copy(
                    rout0.at[pl.ds(0, ZR)],
                    acc.at[pl.ds(sid * DR + z * ZR, ZR)],
                )

            @pl.when(sid == NS - 1)
            def _ztail():
                pltpu.sync_copy(
                    rout0.at[pl.ds(0, TAIL)], acc.at[pl.ds(NS * DR, TAIL)]
                )

            plsc.subcore_barrier()

            @pl.loop(0, NSUP)
            def _sup(sc):
                sbase = ebase + sc * SG
                pltpu.sync_copy(src_hbm.at[pl.ds(sbase, SG)], srcv)
                pltpu.sync_copy(dst_hbm.at[pl.ds(sbase, SG)], dstv)
                pltpu.sync_copy(w_hbm.at[pl.ds(sbase, SG)], wv)

                @pl.loop(0, SG // 16)
                def _rebase(i):
                    gidx[pl.ds(i * 16, 16)] = srcv[pl.ds(i * 16, 16)] + off

                gather(0, 0)

                @pl.loop(0, CPS - 1, step=6)
                def _chunk(ci):
                    for r in range(6):
                        c = ci + r

                        gather(c + 1, (r + 1) % 2)

                        @pl.when(c >= 3)
                        def _ws():
                            wait_scatter(r % 3)

                        process(c, r % 2, r % 3)

                wait_scatter(0)
                process(CPS - 1, 0, 0)
                wait_scatter(1)
                wait_scatter(2)
                wait_scatter(0)

            plsc.subcore_barrier()
            pltpu.sync_copy(
                acc.at[pl.ds(sid * DR, DR)],
                y_hbm.at[pl.ds(off + sid * DR, DR)],
            )

            @pl.when(sid == NS - 1)
            def _dtail():
                pltpu.sync_copy(
                    acc.at[pl.ds(NS * DR, TAIL)],
                    y_hbm.at[pl.ds(off + NS * DR, TAIL)],
                )

    return run(xpairs, src, dst, w)


def _mix_body(x_ref, a_ref, b_ref, wa_ref, wb_ref, wc_ref, bias_ref, o_ref):
    acc = jnp.dot(x_ref[...], wa_ref[...], preferred_element_type=jnp.float32)
    acc = acc + jnp.dot(a_ref[...], wb_ref[...], preferred_element_type=jnp.float32)
    acc = acc + jnp.dot(b_ref[...], wc_ref[...], preferred_element_type=jnp.float32)
    o_ref[...] = acc + bias_ref[...]


def _mix_tc(xflat, s1, s2, wa, wb, wc, bias2):
    TM = 1000
    grid = (B * M // TM,)
    row_spec = pl.BlockSpec((TM, F), lambda i: (i, 0))
    w_spec = pl.BlockSpec((F, F), lambda i: (0, 0))
    return pl.pallas_call(
        _mix_body,
        grid=grid,
        in_specs=[row_spec, row_spec, row_spec, w_spec, w_spec, w_spec,
                  pl.BlockSpec((1, F), lambda i: (0, 0))],
        out_specs=row_spec,
        out_shape=jax.ShapeDtypeStruct((B * M, F), jnp.float32),
    )(xflat, s1, s2, wa, wb, wc, bias2)


def kernel(x, edge_index, edge_weight, kernel, bias):
    xflat = x.reshape(B * M, F)
    src = edge_index[0]
    dst = edge_index[1]

    s1p = _spmm_sc(_to_pairs(xflat), src, dst, edge_weight)
    s2p = _spmm_sc(_to_pairs(s1p), src, dst, edge_weight)

    w3 = kernel.reshape(F, K, -1)
    wa = w3[:, 0, :] - w3[:, 2, :]
    wb = w3[:, 1, :][_PERM]
    wc = (2.0 * w3[:, 2, :])[_PERM2]
    out = _mix_tc(xflat, s1p, s2p, wa, wb, wc, bias.reshape(1, -1))
    return out.reshape(B, M, -1)


# speedup vs baseline: 2.2171x; 2.2171x over previous
import functools

import jax
import jax.numpy as jnp
from jax import lax
from jax.experimental import pallas as pl
from jax.experimental.pallas import tpu as pltpu
from jax.experimental.pallas import tpu_sc as plsc

B, M, F, K, E = 8, 10000, 128, 3, 320000
NC, NS = 2, 16
EP = E // NS
G = 80
SG = 2000
CPS = SG // G
NSUP = EP // SG
DR = 624
TAIL = M - NS * DR
ZR = 52
BPC = B // NC


def _cheb_sc(xflat, src, dst, w):
    mesh = plsc.VectorSubcoreMesh(
        core_axis_name="c", subcore_axis_name="s", num_cores=NC, num_subcores=NS
    )

    @functools.partial(
        pl.kernel,
        out_type=(
            jax.ShapeDtypeStruct((B * M, F), jnp.float32),
            jax.ShapeDtypeStruct((B * M, F), jnp.float32),
        ),
        mesh=mesh,
        scratch_types=[
            pltpu.VMEM_SHARED((M, F), jnp.float32),
            pltpu.VMEM((ZR, F), jnp.float32),
            pltpu.VMEM((SG,), jnp.int32),
            pltpu.VMEM((SG,), jnp.int32),
            pltpu.VMEM((SG,), jnp.float32),
            pltpu.VMEM((SG,), jnp.int32),
            pltpu.VMEM((G, F), jnp.float32),
            pltpu.VMEM((G, F), jnp.float32),
            pltpu.VMEM((G, F), jnp.float32),
            pltpu.VMEM((G,), jnp.int32),
            pltpu.VMEM((G,), jnp.int32),
            pltpu.VMEM((G,), jnp.int32),
            pltpu.SemaphoreType.DMA,
            pltpu.SemaphoreType.DMA,
            pltpu.SemaphoreType.DMA,
            pltpu.SemaphoreType.DMA,
            pltpu.SemaphoreType.DMA,
            pltpu.SemaphoreType.DMA,
        ],
    )
    def run(x_hbm, src_hbm, dst_hbm, w_hbm, y1_hbm, y2_hbm,
            acc, zbuf, srcv, dstv, wv, gidx, rows0, rows1, rows2,
            d0, d1, d2, gsem0, gsem1, gsem2, ssem0, ssem1, ssem2):
        cid = lax.axis_index("c")
        sid = lax.axis_index("s")
        ebase = sid * EP

        rows = (rows0, rows1, rows2)
        dbuf = (d0, d1, d2)
        gsems = (gsem0, gsem1, gsem2)
        ssems = (ssem0, ssem1, ssem2)

        @pl.loop(0, ZR)
        def _zfill(r):
            for j in range(F // 16):
                zbuf[r, pl.ds(j * 16, 16)] = jnp.zeros((16,), jnp.float32)

        def one_pass(in_hbm, out_hbm, off):
            def gather(ci, p):
                pltpu.async_copy(
                    in_hbm.at[gidx.at[pl.ds(ci * G, G)]], rows[p], gsems[p]
                )

            def wait_scatter(p):
                pltpu.make_async_copy(rows[p], acc.at[dbuf[p]], ssems[p]).wait()

            def process(ci, p):
                pltpu.make_async_copy(
                    in_hbm.at[gidx.at[pl.ds(ci * G, G)]], rows[p], gsems[p]
                ).wait()
                for j in range(G // 16):
                    dbuf[p][pl.ds(j * 16, 16)] = dstv[pl.ds(ci * G + j * 16, 16)]

                @pl.loop(0, G // 16)
                def _scale(g):
                    wvec = wv[pl.ds(ci * G + g * 16, 16)]
                    for l in range(16):
                        we = wvec[l]
                        e = g * 16 + l
                        for j in range(F // 16):
                            rows[p][e, pl.ds(j * 16, 16)] = (
                                rows[p][e, pl.ds(j * 16, 16)] * we
                            )

                pltpu.async_copy(rows[p], acc.at[dbuf[p]], ssems[p], add=True)

            for z in range(DR // ZR):
                pltpu.sync_copy(zbuf, acc.at[pl.ds(sid * DR + z * ZR, ZR)])

            @pl.when(sid == NS - 1)
            def _ztail():
                pltpu.sync_copy(zbuf.at[pl.ds(0, TAIL)], acc.at[pl.ds(NS * DR, TAIL)])

            plsc.subcore_barrier()

            @pl.loop(0, NSUP)
            def _sup(sc):
                sbase = ebase + sc * SG
                pltpu.sync_copy(src_hbm.at[pl.ds(sbase, SG)], srcv)
                pltpu.sync_copy(dst_hbm.at[pl.ds(sbase, SG)], dstv)
                pltpu.sync_copy(w_hbm.at[pl.ds(sbase, SG)], wv)

                @pl.loop(0, SG // 16)
                def _rebase(i):
                    gidx[pl.ds(i * 16, 16)] = srcv[pl.ds(i * 16, 16)] + off

                gather(0, 0)

                @pl.loop(0, CPS - 1, step=3)
                def _chunk(ci):
                    for r in range(3):
                        c = ci + r
                        q = (r + 1) % 3

                        @pl.when(c >= 2)
                        def _ws():
                            wait_scatter(q)

                        gather(c + 1, q)
                        process(c, r)

                wait_scatter(1)
                process(CPS - 1, 0)
                wait_scatter(2)
                wait_scatter(0)

            plsc.subcore_barrier()
            pltpu.sync_copy(
                acc.at[pl.ds(sid * DR, DR)],
                out_hbm.at[pl.ds(off + sid * DR, DR)],
            )

            @pl.when(sid == NS - 1)
            def _dtail():
                pltpu.sync_copy(
                    acc.at[pl.ds(NS * DR, TAIL)],
                    out_hbm.at[pl.ds(off + NS * DR, TAIL)],
                )

        @pl.loop(0, BPC)
        def _batch(bi):
            b = cid * BPC + bi
            off = b * M
            one_pass(x_hbm, y1_hbm, off)
            one_pass(y1_hbm, y2_hbm, off)

    return run(xflat, src, dst, w)


def _mix_body(x_ref, a_ref, b_ref, wa_ref, wb_ref, wc_ref, bias_ref, o_ref):
    acc = jnp.dot(x_ref[...], wa_ref[...], preferred_element_type=jnp.float32)
    acc = acc + jnp.dot(a_ref[...], wb_ref[...], preferred_element_type=jnp.float32)
    acc = acc + jnp.dot(b_ref[...], wc_ref[...], preferred_element_type=jnp.float32)
    o_ref[...] = acc + bias_ref[...]


def _mix_tc(xflat, s1, s2, wa, wb, wc, bias2):
    TM = 1000
    grid = (B * M // TM,)
    row_spec = pl.BlockSpec((TM, F), lambda i: (i, 0))
    w_spec = pl.BlockSpec((F, F), lambda i: (0, 0))
    return pl.pallas_call(
        _mix_body,
        grid=grid,
        in_specs=[row_spec, row_spec, row_spec, w_spec, w_spec, w_spec,
                  pl.BlockSpec((1, F), lambda i: (0, 0))],
        out_specs=row_spec,
        out_shape=jax.ShapeDtypeStruct((B * M, F), jnp.float32),
    )(xflat, s1, s2, wa, wb, wc, bias2)


def kernel(x, edge_index, edge_weight, kernel, bias):
    xflat = x.reshape(B * M, F)
    src = edge_index[0]
    dst = edge_index[1]

    s1, s2 = _cheb_sc(xflat, src, dst, edge_weight)

    w3 = kernel.reshape(F, K, -1)
    wa = w3[:, 0, :] - w3[:, 2, :]
    wb = w3[:, 1, :]
    wc = 2.0 * w3[:, 2, :]
    out = _mix_tc(xflat, s1, s2, wa, wb, wc, bias.reshape(1, -1))
    return out.reshape(B, M, -1)


# speedup vs baseline: 2.3330x; 1.0523x over previous
import functools

import jax
import jax.numpy as jnp
from jax import lax
from jax.experimental import pallas as pl
from jax.experimental.pallas import tpu as pltpu
from jax.experimental.pallas import tpu_sc as plsc

B, M, F, K, E = 8, 10000, 128, 3, 320000
NC, NS = 2, 16
EP = E // NS
G = 80
SG = 2000
CPS = SG // G
NSUP = EP // SG
DR = 624
TAIL = M - NS * DR
ZR = 52
BPC = B // NC


def _cheb_sc(xflat, src, dst, w):
    mesh = plsc.VectorSubcoreMesh(
        core_axis_name="c", subcore_axis_name="s", num_cores=NC, num_subcores=NS
    )

    @functools.partial(
        pl.kernel,
        out_type=(
            jax.ShapeDtypeStruct((B * M, F), jnp.float32),
            jax.ShapeDtypeStruct((B * M, F), jnp.float32),
        ),
        mesh=mesh,
        scratch_types=[
            pltpu.VMEM_SHARED((M, F), jnp.float32),
            pltpu.VMEM((ZR, F), jnp.float32),
            pltpu.VMEM((SG,), jnp.int32),
            pltpu.VMEM((SG,), jnp.int32),
            pltpu.VMEM((SG,), jnp.float32),
            pltpu.VMEM((SG,), jnp.int32),
            pltpu.VMEM((SG,), jnp.int32),
            pltpu.VMEM((SG,), jnp.float32),
            pltpu.VMEM((G, F), jnp.float32),
            pltpu.VMEM((G, F), jnp.float32),
            pltpu.VMEM((G, F), jnp.float32),
            pltpu.VMEM((G,), jnp.int32),
            pltpu.VMEM((G,), jnp.int32),
            pltpu.VMEM((G,), jnp.int32),
            pltpu.SemaphoreType.DMA,
            pltpu.SemaphoreType.DMA,
            pltpu.SemaphoreType.DMA,
            pltpu.SemaphoreType.DMA,
            pltpu.SemaphoreType.DMA,
            pltpu.SemaphoreType.DMA,
            pltpu.SemaphoreType.DMA,
            pltpu.SemaphoreType.DMA,
        ],
    )
    def run(x_hbm, src_hbm, dst_hbm, w_hbm, y1_hbm, y2_hbm,
            acc, zbuf, gidxA, dstA, wA, gidxB, dstB, wB,
            rows0, rows1, rows2, d0, d1, d2,
            gsem0, gsem1, gsem2, ssem0, ssem1, ssem2, stsemA, stsemB):
        cid = lax.axis_index("c")
        sid = lax.axis_index("s")
        ebase = sid * EP

        setA = (gidxA, dstA, wA, stsemA)
        setB = (gidxB, dstB, wB, stsemB)
        rows = (rows0, rows1, rows2)
        dbuf = (d0, d1, d2)
        gsems = (gsem0, gsem1, gsem2)
        ssems = (ssem0, ssem1, ssem2)

        @pl.loop(0, ZR)
        def _zfill(r):
            for j in range(F // 16):
                zbuf[r, pl.ds(j * 16, 16)] = jnp.zeros((16,), jnp.float32)

        def stage(si, sset):
            gidxS, dstS, wS, sem = sset
            sbase = ebase + si * SG
            pltpu.async_copy(src_hbm.at[pl.ds(sbase, SG)], gidxS, sem)
            pltpu.async_copy(dst_hbm.at[pl.ds(sbase, SG)], dstS, sem)
            pltpu.async_copy(w_hbm.at[pl.ds(sbase, SG)], wS, sem)

        def wait_stage_rebase(si, sset, off):
            gidxS, dstS, wS, sem = sset
            sbase = ebase + si * SG
            pltpu.make_async_copy(src_hbm.at[pl.ds(sbase, SG)], gidxS, sem).wait()
            pltpu.make_async_copy(dst_hbm.at[pl.ds(sbase, SG)], dstS, sem).wait()
            pltpu.make_async_copy(w_hbm.at[pl.ds(sbase, SG)], wS, sem).wait()

            @pl.loop(0, SG // 16)
            def _rebase(i):
                gidxS[pl.ds(i * 16, 16)] = gidxS[pl.ds(i * 16, 16)] + off

        def do_sup(pp, sset):
            gidxS, dstS, wS, _ = sset

            def gather(ci, p):
                idx = gidxS.at[pl.ds(ci * G, G)]

                @pl.when(pp == 0)
                def _g0():
                    pltpu.async_copy(x_hbm.at[idx], rows[p], gsems[p])

                @pl.when(pp == 1)
                def _g1():
                    pltpu.async_copy(y1_hbm.at[idx], rows[p], gsems[p])

            def wait_gather(ci, p):
                pltpu.make_async_copy(
                    x_hbm.at[gidxS.at[pl.ds(ci * G, G)]], rows[p], gsems[p]
                ).wait()

            def wait_scatter(p):
                pltpu.make_async_copy(rows[p], acc.at[dbuf[p]], ssems[p]).wait()

            def process(ci, p):
                wait_gather(ci, p)
                for j in range(G // 16):
                    dbuf[p][pl.ds(j * 16, 16)] = dstS[pl.ds(ci * G + j * 16, 16)]

                @pl.loop(0, G // 16)
                def _scale(g):
                    wvec = wS[pl.ds(ci * G + g * 16, 16)]
                    for l in range(16):
                        we = wvec[l]
                        e = g * 16 + l
                        for j in range(F // 16):
                            rows[p][e, pl.ds(j * 16, 16)] = (
                                rows[p][e, pl.ds(j * 16, 16)] * we
                            )

                pltpu.async_copy(rows[p], acc.at[dbuf[p]], ssems[p], add=True)

            gather(0, 0)

            @pl.loop(0, CPS - 1, step=3)
            def _chunk(ci):
                for r in range(3):
                    c = ci + r
                    q = (r + 1) % 3

                    @pl.when(c >= 2)
                    def _ws():
                        wait_scatter(q)

                    gather(c + 1, q)
                    process(c, r)

            wait_scatter(1)
            process(CPS - 1, 0)
            wait_scatter(2)
            wait_scatter(0)

        def one_pass(pp, off):
            for z in range(DR // ZR):
                pltpu.sync_copy(zbuf, acc.at[pl.ds(sid * DR + z * ZR, ZR)])

            @pl.when(sid == NS - 1)
            def _ztail():
                pltpu.sync_copy(zbuf.at[pl.ds(0, TAIL)], acc.at[pl.ds(NS * DR, TAIL)])

            plsc.subcore_barrier()

            stage(0, setA)
            wait_stage_rebase(0, setA, off)

            @pl.loop(0, NSUP, step=2)
            def _pair(si):
                stage(si + 1, setB)
                do_sup(pp, setA)
                wait_stage_rebase(si + 1, setB, off)

                @pl.when(si + 2 < NSUP)
                def _sA():
                    stage(si + 2, setA)

                do_sup(pp, setB)

                @pl.when(si + 2 < NSUP)
                def _wA():
                    wait_stage_rebase(si + 2, setA, off)

            plsc.subcore_barrier()

            @pl.when(pp == 0)
            def _d0():
                pltpu.sync_copy(
                    acc.at[pl.ds(sid * DR, DR)],
                    y1_hbm.at[pl.ds(off + sid * DR, DR)],
                )

                @pl.when(sid == NS - 1)
                def _dtail0():
                    pltpu.sync_copy(
                        acc.at[pl.ds(NS * DR, TAIL)],
                        y1_hbm.at[pl.ds(off + NS * DR, TAIL)],
                    )

            @pl.when(pp == 1)
            def _d1():
                pltpu.sync_copy(
                    acc.at[pl.ds(sid * DR, DR)],
                    y2_hbm.at[pl.ds(off + sid * DR, DR)],
                )

                @pl.when(sid == NS - 1)
                def _dtail1():
                    pltpu.sync_copy(
                        acc.at[pl.ds(NS * DR, TAIL)],
                        y2_hbm.at[pl.ds(off + NS * DR, TAIL)],
                    )

        @pl.loop(0, BPC)
        def _batch(bi):
            b = cid * BPC + bi
            off = b * M

            @pl.loop(0, 2)
            def _pass(pp):
                one_pass(pp, off)

    return run(xflat, src, dst, w)


def _mix_body(x_ref, a_ref, b_ref, wa_ref, wb_ref, wc_ref, bias_ref, o_ref):
    acc = jnp.dot(x_ref[...], wa_ref[...], preferred_element_type=jnp.float32)
    acc = acc + jnp.dot(a_ref[...], wb_ref[...], preferred_element_type=jnp.float32)
    acc = acc + jnp.dot(b_ref[...], wc_ref[...], preferred_element_type=jnp.float32)
    o_ref[...] = acc + bias_ref[...]


def _mix_tc(xflat, s1, s2, wa, wb, wc, bias2):
    TM = 1000
    grid = (B * M // TM,)
    row_spec = pl.BlockSpec((TM, F), lambda i: (i, 0))
    w_spec = pl.BlockSpec((F, F), lambda i: (0, 0))
    return pl.pallas_call(
        _mix_body,
        grid=grid,
        in_specs=[row_spec, row_spec, row_spec, w_spec, w_spec, w_spec,
                  pl.BlockSpec((1, F), lambda i: (0, 0))],
        out_specs=row_spec,
        out_shape=jax.ShapeDtypeStruct((B * M, F), jnp.float32),
    )(xflat, s1, s2, wa, wb, wc, bias2)


def kernel(x, edge_index, edge_weight, kernel, bias):
    xflat = x.reshape(B * M, F)
    src = edge_index[0]
    dst = edge_index[1]

    s1, s2 = _cheb_sc(xflat, src, dst, edge_weight)

    w3 = kernel.reshape(F, K, -1)
    wa = w3[:, 0, :] - w3[:, 2, :]
    wb = w3[:, 1, :]
    wc = 2.0 * w3[:, 2, :]
    out = _mix_tc(xflat, s1, s2, wa, wb, wc, bias.reshape(1, -1))
    return out.reshape(B, M, -1)
